# Initial kernel scaffold; baseline (speedup 1.0000x reference)
import jax
import jax.numpy as jnp
from jax.experimental import pallas as pl


def kernel(candidate_log_probs, candidate_to_sample_idx, sample_has_bug, sample_to_correct_candidate_idx, sample_is_nonpad, train_step):
    raise NotImplementedError("write your pallas kernel here")


# speedup vs baseline: 17.0427x; 17.0427x over previous
import math

import jax
import jax.numpy as jnp
from jax import lax
from jax.experimental import pallas as pl
from jax.experimental.pallas import tpu as pltpu
from jax.experimental.pallas import tpu_sc as plsc

_ABSTAIN_WEIGHT = 0.1
_NUM_WORKERS = 32
_LANES = 16


def _sc_body(nc, ns, piece, npiece,
             clp_hbm, idx_hbm, s2c_hbm,
             partmax_hbm, lpc_hbm,
             lmax_v, val_v, idx_v, gidx_v, gout_v, sem):
    chunk = nc // _NUM_WORKERS
    sub = piece // _LANES
    spw = ns // _NUM_WORKERS
    wid = lax.axis_index("s") * 2 + lax.axis_index("c")

    lane = lax.iota(jnp.int32, _LANES)
    ninf = jnp.full((_LANES,), -jnp.inf, jnp.float32)

    def _init(i, _):
        lmax_v[pl.ds(i * _LANES, _LANES)] = ninf
        return 0
    lax.fori_loop(0, ns // _LANES, _init, 0)

    cbase = wid * chunk
    for p in range(npiece):
        pbase = cbase + p * piece
        pltpu.sync_copy(idx_hbm.at[pl.ds(pbase, piece)], idx_v)
        pltpu.sync_copy(clp_hbm.at[pl.ds(pbase, piece)], val_v)

        lane_base = lane * sub
        hs = plsc.load_gather(idx_v, [lane_base])

        def _rake(t, carry):
            m, cs, is_head, head_max = carry
            offs = lane_base + t
            v = plsc.load_gather(val_v, [offs])
            s = plsc.load_gather(idx_v, [offs])
            changed = s != cs
            flush = jnp.logical_and(changed, jnp.logical_not(is_head))
            plsc.store_scatter(lmax_v, [cs], m, mask=flush)
            head_max = jnp.where(jnp.logical_and(changed, is_head), m, head_max)
            is_head = jnp.logical_and(is_head, jnp.logical_not(changed))
            m = jnp.where(changed, v, jnp.maximum(m, v))
            return m, s, is_head, head_max

        m, cs, is_head, head_max = lax.fori_loop(
            0, sub, _rake,
            (ninf, hs, jnp.full((_LANES,), True), ninf))

        head_final = jnp.where(is_head, m, head_max)
        for j in range(_LANES):
            g = plsc.load_gather(lmax_v, [hs])
            plsc.store_scatter(lmax_v, [hs], jnp.maximum(g, head_final),
                               mask=lane == j)
        for j in range(_LANES):
            g = plsc.load_gather(lmax_v, [cs])
            plsc.store_scatter(lmax_v, [cs], jnp.maximum(g, m),
                               mask=lane == j)

    pltpu.sync_copy(lmax_v, partmax_hbm.at[wid])

    sbase = wid * spw
    pltpu.sync_copy(s2c_hbm.at[pl.ds(sbase, spw)], gidx_v)
    pltpu.async_copy(clp_hbm.at[gidx_v], gout_v, sem).wait()
    pltpu.sync_copy(gout_v, lpc_hbm.at[pl.ds(sbase, spw)])


def _tc_body(pm_ref, tail_ref, lpc_ref, bug_ref, np_ref,
             loss_ref, nnp_ref, corr_ref, nb_ref, nbc_ref):
    pm = pm_ref[...]
    tail = tail_ref[...]
    lpc = lpc_ref[...]
    bug = bug_ref[...]
    nonpad = np_ref[...]

    is_bug = bug == 1
    is_np = nonpad == 1

    seg_max = jnp.maximum(jnp.max(pm, axis=0), tail)
    lp = jnp.where(is_bug, lpc, tail)
    lp = jnp.where(is_np, lp, 0.0)
    lp = jnp.minimum(lp, math.log(0.995))
    lp = lp + jnp.where(jnp.logical_and(is_bug, is_np),
                        _ABSTAIN_WEIGHT * tail,
                        jnp.zeros_like(lp))

    nnp = jnp.sum(nonpad)
    correct = jnp.logical_and(seg_max == lp, is_np)
    nobug = jnp.logical_and(jnp.logical_not(is_bug), is_np)

    loss_ref[0, 0] = -jnp.sum(lp) / nnp.astype(jnp.float32)
    nnp_ref[0, 0] = nnp
    corr_ref[0, 0] = jnp.sum(correct.astype(jnp.int32))
    nb_ref[0, 0] = jnp.sum(nobug.astype(jnp.int32))
    nbc_ref[0, 0] = jnp.sum(jnp.logical_and(nobug, correct).astype(jnp.int32))


def kernel(candidate_log_probs, candidate_to_sample_idx, sample_has_bug,
           sample_to_correct_candidate_idx, sample_is_nonpad, train_step):
    nc = candidate_to_sample_idx.shape[0]
    ns = sample_has_bug.shape[0]
    npiece = 4
    piece = nc // (_NUM_WORKERS * npiece)

    sc_fn = pl.kernel(
        lambda *refs: _sc_body(nc, ns, piece, npiece, *refs),
        out_type=(
            jax.ShapeDtypeStruct((_NUM_WORKERS, ns), jnp.float32),
            jax.ShapeDtypeStruct((ns,), jnp.float32),
        ),
        mesh=plsc.VectorSubcoreMesh(core_axis_name="c", subcore_axis_name="s"),
        compiler_params=pltpu.CompilerParams(needs_layout_passes=False),
        scratch_types=[
            pltpu.VMEM((ns,), jnp.float32),
            pltpu.VMEM((piece,), jnp.float32),
            pltpu.VMEM((piece,), jnp.int32),
            pltpu.VMEM((ns // _NUM_WORKERS,), jnp.int32),
            pltpu.VMEM((ns // _NUM_WORKERS,), jnp.float32),
            pltpu.SemaphoreType.DMA,
        ],
    )
    partmax, lpc = sc_fn(candidate_log_probs,
                         candidate_to_sample_idx,
                         sample_to_correct_candidate_idx)

    srows = ns // 128
    outs = pl.pallas_call(
        _tc_body,
        out_shape=(
            jax.ShapeDtypeStruct((1, 1), jnp.float32),
            jax.ShapeDtypeStruct((1, 1), jnp.int32),
            jax.ShapeDtypeStruct((1, 1), jnp.int32),
            jax.ShapeDtypeStruct((1, 1), jnp.int32),
            jax.ShapeDtypeStruct((1, 1), jnp.int32),
        ),
        out_specs=tuple(pl.BlockSpec(memory_space=pltpu.SMEM)
                        for _ in range(5)),
    )(
        partmax.reshape(_NUM_WORKERS, srows, 128),
        candidate_log_probs[nc:].reshape(srows, 128),
        lpc.reshape(srows, 128),
        sample_has_bug.astype(jnp.int32).reshape(srows, 128),
        sample_is_nonpad.astype(jnp.int32).reshape(srows, 128),
    )
    loss, nnp, corr, nb, nbc = outs
    return (loss.reshape(()), nnp.reshape(()), corr.reshape(()),
            nb.reshape(()), nbc.reshape(()))


# speedup vs baseline: 28.6317x; 1.6800x over previous
import math

import jax
import jax.numpy as jnp
from jax import lax
from jax.experimental import pallas as pl
from jax.experimental.pallas import tpu as pltpu
from jax.experimental.pallas import tpu_sc as plsc

_ABSTAIN_WEIGHT = 0.1
_NUM_WORKERS = 32
_LANES = 16


def _sc_body(nc, ns, piece, npiece,
             clp_hbm, idx_hbm, s2c_hbm,
             partmax_hbm, lpc_hbm,
             lmax_v, val0_v, val1_v, idx0_v, idx1_v, gidx_v, gout_v,
             sem0, sem1, sem):
    chunk = nc // _NUM_WORKERS
    sub = piece // _LANES
    spw = ns // _NUM_WORKERS
    wid = lax.axis_index("s") * 2 + lax.axis_index("c")

    lane = lax.iota(jnp.int32, _LANES)
    ninf = jnp.full((_LANES,), -jnp.inf, jnp.float32)

    val_bufs = (val0_v, val1_v)
    idx_bufs = (idx0_v, idx1_v)
    sems = (sem0, sem1)

    cbase = wid * chunk

    def _start(p):
        pbase = cbase + p * piece
        b = p % 2
        return (
            pltpu.async_copy(idx_hbm.at[pl.ds(pbase, piece)], idx_bufs[b],
                             sems[b]),
            pltpu.async_copy(clp_hbm.at[pl.ds(pbase, piece)], val_bufs[b],
                             sems[b]),
        )

    pending = _start(0)

    @plsc.parallel_loop(0, ns // _LANES, unroll=8)
    def _init(i):
        lmax_v[pl.ds(i * _LANES, _LANES)] = ninf

    for p in range(npiece):
        b = p % 2
        val_v, idx_v = val_bufs[b], idx_bufs[b]
        nxt = _start(p + 1) if p + 1 < npiece else ()
        for d in pending:
            d.wait()
        pending = nxt

        lane_base = lane * sub
        hs = plsc.load_gather(idx_v, [lane_base])

        def _rake(t, carry):
            m, cs, is_head, head_max = carry
            offs = lane_base + t
            v = plsc.load_gather(val_v, [offs])
            s = plsc.load_gather(idx_v, [offs])
            changed = s != cs
            flush = jnp.logical_and(changed, jnp.logical_not(is_head))
            plsc.store_scatter(lmax_v, [cs], m, mask=flush)
            head_max = jnp.where(jnp.logical_and(changed, is_head), m, head_max)
            is_head = jnp.logical_and(is_head, jnp.logical_not(changed))
            m = jnp.where(changed, v, jnp.maximum(m, v))
            return m, s, is_head, head_max

        m, cs, is_head, head_max = plsc.parallel_loop(
            0, sub, unroll=4,
            carry=(ninf, hs, jnp.full((_LANES,), True), ninf))(_rake)

        head_final = jnp.where(is_head, m, head_max)
        for j in range(_LANES):
            g = plsc.load_gather(lmax_v, [hs])
            plsc.store_scatter(lmax_v, [hs], jnp.maximum(g, head_final),
                               mask=lane == j)
        for j in range(_LANES):
            g = plsc.load_gather(lmax_v, [cs])
            plsc.store_scatter(lmax_v, [cs], jnp.maximum(g, m),
                               mask=lane == j)

    pltpu.sync_copy(lmax_v, partmax_hbm.at[wid])

    sbase = wid * spw
    pltpu.sync_copy(s2c_hbm.at[pl.ds(sbase, spw)], gidx_v)
    pltpu.async_copy(clp_hbm.at[gidx_v], gout_v, sem).wait()
    pltpu.sync_copy(gout_v, lpc_hbm.at[pl.ds(sbase, spw)])


def _tc_body(pm_ref, tail_ref, lpc_ref, bug_ref, np_ref,
             loss_ref, nnp_ref, corr_ref, nb_ref, nbc_ref):
    pm = pm_ref[...]
    tail = tail_ref[...]
    lpc = lpc_ref[...]
    bug = bug_ref[...]
    nonpad = np_ref[...]

    is_bug = bug == 1
    is_np = nonpad == 1

    seg_max = jnp.maximum(jnp.max(pm, axis=0), tail)
    lp = jnp.where(is_bug, lpc, tail)
    lp = jnp.where(is_np, lp, 0.0)
    lp = jnp.minimum(lp, math.log(0.995))
    lp = lp + jnp.where(jnp.logical_and(is_bug, is_np),
                        _ABSTAIN_WEIGHT * tail,
                        jnp.zeros_like(lp))

    nnp = jnp.sum(nonpad)
    correct = jnp.logical_and(seg_max == lp, is_np)
    nobug = jnp.logical_and(jnp.logical_not(is_bug), is_np)

    loss_ref[0, 0] = -jnp.sum(lp) / nnp.astype(jnp.float32)
    nnp_ref[0, 0] = nnp
    corr_ref[0, 0] = jnp.sum(correct.astype(jnp.int32))
    nb_ref[0, 0] = jnp.sum(nobug.astype(jnp.int32))
    nbc_ref[0, 0] = jnp.sum(jnp.logical_and(nobug, correct).astype(jnp.int32))


def kernel(candidate_log_probs, candidate_to_sample_idx, sample_has_bug,
           sample_to_correct_candidate_idx, sample_is_nonpad, train_step):
    nc = candidate_to_sample_idx.shape[0]
    ns = sample_has_bug.shape[0]
    npiece = 4
    piece = nc // (_NUM_WORKERS * npiece)

    sc_fn = pl.kernel(
        lambda *refs: _sc_body(nc, ns, piece, npiece, *refs),
        out_type=(
            jax.ShapeDtypeStruct((_NUM_WORKERS, ns), jnp.float32),
            jax.ShapeDtypeStruct((ns,), jnp.float32),
        ),
        mesh=plsc.VectorSubcoreMesh(core_axis_name="c", subcore_axis_name="s"),
        compiler_params=pltpu.CompilerParams(needs_layout_passes=False),
        scratch_types=[
            pltpu.VMEM((ns,), jnp.float32),
            pltpu.VMEM((piece,), jnp.float32),
            pltpu.VMEM((piece,), jnp.float32),
            pltpu.VMEM((piece,), jnp.int32),
            pltpu.VMEM((piece,), jnp.int32),
            pltpu.VMEM((ns // _NUM_WORKERS,), jnp.int32),
            pltpu.VMEM((ns // _NUM_WORKERS,), jnp.float32),
            pltpu.SemaphoreType.DMA,
            pltpu.SemaphoreType.DMA,
            pltpu.SemaphoreType.DMA,
        ],
    )
    partmax, lpc = sc_fn(candidate_log_probs,
                         candidate_to_sample_idx,
                         sample_to_correct_candidate_idx)

    srows = ns // 128
    outs = pl.pallas_call(
        _tc_body,
        out_shape=(
            jax.ShapeDtypeStruct((1, 1), jnp.float32),
            jax.ShapeDtypeStruct((1, 1), jnp.int32),
            jax.ShapeDtypeStruct((1, 1), jnp.int32),
            jax.ShapeDtypeStruct((1, 1), jnp.int32),
            jax.ShapeDtypeStruct((1, 1), jnp.int32),
        ),
        out_specs=tuple(pl.BlockSpec(memory_space=pltpu.SMEM)
                        for _ in range(5)),
    )(
        partmax.reshape(_NUM_WORKERS, srows, 128),
        candidate_log_probs[nc:].reshape(srows, 128),
        lpc.reshape(srows, 128),
        sample_has_bug.astype(jnp.int32).reshape(srows, 128),
        sample_is_nonpad.astype(jnp.int32).reshape(srows, 128),
    )
    loss, nnp, corr, nb, nbc = outs
    return (loss.reshape(()), nnp.reshape(()), corr.reshape(()),
            nb.reshape(()), nbc.reshape(()))
